# Initial kernel scaffold; baseline (speedup 1.0000x reference)
import jax
import jax.numpy as jnp
from jax.experimental import pallas as pl


def kernel(vertices, w0, d0, w1, b1, disp1, w2, b2, disp2):
    raise NotImplementedError("write your pallas kernel here")


# speedup vs baseline: 1.5630x; 1.5630x over previous
import functools
import math

import jax
import jax.numpy as jnp
from jax.experimental import pallas as pl
from jax.experimental.pallas import tpu as pltpu

NEIGHBOR_NUM = 20
KERNEL_NUM = 32
MID_CH = 64
LOCAL_LATENT = 128


def _topk_body(v_ref, vt_ref, out_ref, dist_ref, *, n, k):
    rows = v_ref.shape[1]
    vr = v_ref[0]
    vt = vt_ref[0]
    inner = jax.lax.dot_general(
        vr, vt, (((1,), (0,)), ((), ())), preferred_element_type=jnp.float32)
    qr = jnp.sum(vr * vr, axis=1, keepdims=True)
    qc = jnp.sum(vt * vt, axis=0, keepdims=True)
    dist_ref[...] = inner * (-2.0) + qc + qr
    col = jax.lax.broadcasted_iota(jnp.int32, (rows, n), 1)
    for t in range(k):
        dist = dist_ref[...]
        m = jnp.min(dist, axis=1, keepdims=True)
        idxt = jnp.min(jnp.where(dist == m, col, n), axis=1, keepdims=True)
        out_ref[0, :, t:t + 1] = idxt
        dist_ref[...] = jnp.where(col == idxt, jnp.inf, dist)


def _knn_topk(vertices, k, rows):
    b, n, _ = vertices.shape
    vt = jnp.transpose(vertices, (0, 2, 1))
    kp = max(8, k)
    out = pl.pallas_call(
        functools.partial(_topk_body, n=n, k=k),
        grid=(b, n // rows),
        in_specs=[
            pl.BlockSpec((1, rows, 3), lambda bi, ri: (bi, ri, 0)),
            pl.BlockSpec((1, 3, n), lambda bi, ri: (bi, 0, 0)),
        ],
        out_specs=pl.BlockSpec((1, rows, kp), lambda bi, ri: (bi, ri, 0)),
        out_shape=jax.ShapeDtypeStruct((b, n, kp), jnp.int32),
        scratch_shapes=[pltpu.VMEM((rows, n), jnp.float32)],
    )(vertices, vt)
    return out[:, :, :k]


def _gather(t, idx):
    return jax.vmap(lambda tt, ii: tt[ii])(t, idx)


def kernel(vertices, w0, d0, w1, b1, disp1, w2, b2, disp2):
    b, v, _ = vertices.shape
    nn = NEIGHBOR_NUM

    idx21 = _knn_topk(vertices, nn + 1, 256)
    nidx = idx21[:, :, 1:]

    disp = _gather(vertices, nidx) - vertices[:, :, None, :]
    theta = jax.nn.relu(disp @ d0)
    fm0 = jnp.max(theta, axis=2) * w0.reshape(1, 1, -1)
    fm0 = jax.nn.relu(fm0)

    theta1 = jax.nn.relu(disp @ disp1)
    fout = fm0 @ w1 + b1
    center, support = fout[..., :MID_CH], fout[..., MID_CH:]
    fm1 = jax.nn.relu(center + jnp.max(theta1 * _gather(support, nidx), axis=2))

    pool_num = v // 8
    sample_idx = jax.random.permutation(jax.random.key(101), v)[:pool_num]
    v1 = vertices[:, sample_idx]
    fm1p = jnp.max(_gather(fm1, nidx[:, :, :8][:, sample_idx]), axis=2)

    nidx2 = _knn_topk(v1, nn + 1, pool_num)[:, :, 1:]

    disp2g = _gather(v1, nidx2) - v1[:, :, None, :]
    theta2 = jax.nn.relu(disp2g @ disp2)
    fout2 = fm1p @ w2 + b2
    center2, support2 = fout2[..., :LOCAL_LATENT], fout2[..., LOCAL_LATENT:]
    fm2 = jax.nn.relu(
        center2 + jnp.max(theta2 * _gather(support2, nidx2), axis=2))

    pool2 = pool_num // 16
    sample2 = jax.random.permutation(jax.random.key(202), pool_num)[:pool2]
    v2 = v1[:, sample2]
    fm3 = jnp.max(_gather(fm2, nidx2[:, :, :16][:, sample2]), axis=2)
    return (v2, fm3)


# speedup vs baseline: 13.7064x; 8.7691x over previous
import functools

import jax
import jax.numpy as jnp
from jax import lax
from jax.experimental import pallas as pl
from jax.experimental.pallas import tpu as pltpu
from jax.experimental.pallas import tpu_sc as plsc

NEIGHBOR_NUM = 20

_NC, _NS, _L = 2, 16, 16
_NW = _NC * _NS

_MESH = plsc.VectorSubcoreMesh(core_axis_name="c", subcore_axis_name="s")


def _wid():
    return lax.axis_index("s") * _NC + lax.axis_index("c")


def _topk_body(v_ref, vt_ref, out_ref, dist_ref, *, n, k):
    rows = v_ref.shape[1]
    vr = v_ref[0]
    vt = vt_ref[0]
    inner = lax.dot_general(
        vr, vt, (((1,), (0,)), ((), ())), preferred_element_type=jnp.float32)
    qr = jnp.sum(vr * vr, axis=1, keepdims=True)
    qc = jnp.sum(vt * vt, axis=0, keepdims=True)
    dist_ref[...] = inner * (-2.0) + qc + qr
    col = lax.broadcasted_iota(jnp.int32, (rows, n), 1)
    for t in range(k):
        dist = dist_ref[...]
        m = jnp.min(dist, axis=1, keepdims=True)
        idxt = jnp.min(jnp.where(dist == m, col, n), axis=1, keepdims=True)
        out_ref[0, :, t:t + 1] = idxt
        dist_ref[...] = jnp.where(col == idxt, jnp.inf, dist)


def _knn_topk(vertices, k, rows):
    b, n, _ = vertices.shape
    vt = jnp.transpose(vertices, (0, 2, 1))
    return pl.pallas_call(
        functools.partial(_topk_body, n=n, k=k),
        grid=(b, n // rows),
        in_specs=[
            pl.BlockSpec((1, rows, 3), lambda bi, ri: (bi, ri, 0)),
            pl.BlockSpec((1, 3, n), lambda bi, ri: (bi, 0, 0)),
        ],
        out_specs=pl.BlockSpec((1, rows, k), lambda bi, ri: (bi, ri, 0)),
        out_shape=jax.ShapeDtypeStruct((b, n, k), jnp.int32),
        scratch_shapes=[pltpu.VMEM((rows, n), jnp.float32)],
    )(vertices, vt)


def _mm(a, b):
    return lax.dot_general(a, b, (((1,), (0,)), ((), ())),
                           preferred_element_type=jnp.float32)


def _dense1(vf, d0, disp1):
    r = vf.shape[0]
    blk = 2048

    def body(v_ref, d0_ref, dp_ref, tb_ref, p64_ref):
        v = v_ref[...]
        p64 = _mm(v, dp_ref[...])
        tb_ref[:, :32] = _mm(v, d0_ref[...])
        tb_ref[:, 32:96] = p64
        tb_ref[:, 96:] = jnp.zeros((blk, 32), jnp.float32)
        p64_ref[...] = p64

    return pl.pallas_call(
        body, grid=(r // blk,),
        in_specs=[pl.BlockSpec((blk, 3), lambda i: (i, 0)),
                  pl.BlockSpec((3, 32), lambda i: (0, 0)),
                  pl.BlockSpec((3, 64), lambda i: (0, 0))],
        out_specs=[pl.BlockSpec((blk, 128), lambda i: (i, 0)),
                   pl.BlockSpec((blk, 64), lambda i: (i, 0))],
        out_shape=[jax.ShapeDtypeStruct((r, 128), jnp.float32),
                   jax.ShapeDtypeStruct((r, 64), jnp.float32)],
    )(vf, d0, disp1)


def _dense2(p, fm, w, bvec, oc):
    r = p.shape[0]
    blk = min(r, 2048)
    wname = 2 * oc

    def body(p_ref, f_ref, w_ref, b_ref, tg_ref, own_ref):
        fout = _mm(f_ref[...], w_ref[...]) + b_ref[...]
        pv = p_ref[...]
        tg_ref[:, :oc] = pv
        tg_ref[:, oc:] = fout[:, oc:]
        own_ref[:, :oc] = pv
        own_ref[:, oc:] = fout[:, :oc]

    return pl.pallas_call(
        body, grid=(r // blk,),
        in_specs=[pl.BlockSpec((blk, oc), lambda i: (i, 0)),
                  pl.BlockSpec((blk, fm.shape[1]), lambda i: (i, 0)),
                  pl.BlockSpec((fm.shape[1], wname), lambda i: (0, 0)),
                  pl.BlockSpec((1, wname), lambda i: (0, 0))],
        out_specs=[pl.BlockSpec((blk, wname), lambda i: (i, 0)),
                   pl.BlockSpec((blk, wname), lambda i: (i, 0))],
        out_shape=[jax.ShapeDtypeStruct((r, wname), jnp.float32),
                   jax.ShapeDtypeStruct((r, wname), jnp.float32)],
    )(p, fm, w, bvec)


def _sc_fm0(t0, gidx, w0v):
    r_tot = t0.shape[0]
    c, n = 4, NEIGHBOR_NUM
    rows_per = r_tot // _NW

    @functools.partial(
        pl.kernel, mesh=_MESH,
        out_type=jax.ShapeDtypeStruct((r_tot, 32), jnp.float32),
        scratch_types=[pltpu.VMEM((c * n,), jnp.int32),
                       pltpu.VMEM((c * n, 128), jnp.float32),
                       pltpu.VMEM((c, 128), jnp.float32),
                       pltpu.VMEM((c, 32), jnp.float32),
                       pltpu.VMEM((32,), jnp.float32),
                       pltpu.SemaphoreType.DMA])
    def k(t_hbm, gidx_hbm, w0_hbm, out_hbm, idxv, gath, own, outv, w0s, sem):
        base0 = _wid() * rows_per
        pltpu.sync_copy(w0_hbm, w0s)
        w0a = w0s[pl.ds(0, _L)]
        w0b = w0s[pl.ds(_L, _L)]

        def chunk(ci, _):
            base = base0 + ci * c
            pltpu.sync_copy(gidx_hbm.at[pl.ds(base * n, c * n)], idxv)
            pltpu.async_copy(t_hbm.at[idxv], gath, sem).wait()
            pltpu.sync_copy(t_hbm.at[pl.ds(base, c)], own)
            for r in range(c):
                o0 = own[r, pl.ds(0, _L)]
                o1 = own[r, pl.ds(_L, _L)]

                def nb(j, acc):
                    row = r * n + j
                    g0 = gath[row, pl.ds(0, _L)]
                    g1 = gath[row, pl.ds(_L, _L)]
                    return (jnp.maximum(acc[0], jnp.maximum(g0 - o0, 0.0)),
                            jnp.maximum(acc[1], jnp.maximum(g1 - o1, 0.0)))

                z = jnp.zeros((_L,), jnp.float32)
                a0, a1 = lax.fori_loop(0, n, nb, (z, z))
                outv[r, pl.ds(0, _L)] = jnp.maximum(a0 * w0a, 0.0)
                outv[r, pl.ds(_L, _L)] = jnp.maximum(a1 * w0b, 0.0)
            pltpu.sync_copy(outv, out_hbm.at[pl.ds(base, c)])
            return 0

        lax.fori_loop(0, rows_per // c, chunk, 0)

    return k(t0, gidx, w0v)


def _sc_edge(tg, gidx, ownt, *, nv):
    r_tot = tg.shape[0]
    w = 2 * nv * _L
    wout = max(nv * _L, 128)
    c, n = 4, NEIGHBOR_NUM
    rows_per = r_tot // _NW

    @functools.partial(
        pl.kernel, mesh=_MESH,
        out_type=jax.ShapeDtypeStruct((r_tot, wout), jnp.float32),
        scratch_types=[pltpu.VMEM((c * n,), jnp.int32),
                       pltpu.VMEM((c * n, w), jnp.float32),
                       pltpu.VMEM((c, w), jnp.float32),
                       pltpu.VMEM((c, wout), jnp.float32),
                       pltpu.SemaphoreType.DMA])
    def k(tg_hbm, gidx_hbm, ownt_hbm, out_hbm, idxv, gath, own, outv, sem):
        base0 = _wid() * rows_per

        def chunk(ci, _):
            base = base0 + ci * c
            pltpu.sync_copy(gidx_hbm.at[pl.ds(base * n, c * n)], idxv)
            pltpu.async_copy(tg_hbm.at[idxv], gath, sem).wait()
            pltpu.sync_copy(ownt_hbm.at[pl.ds(base, c)], own)
            for r in range(c):
                op = [own[r, pl.ds(kk * _L, _L)] for kk in range(nv)]

                def nb(j, acc):
                    row = r * n + j
                    res = []
                    for kk in range(nv):
                        gp = gath[row, pl.ds(kk * _L, _L)]
                        gs = gath[row, pl.ds((nv + kk) * _L, _L)]
                        th = jnp.maximum(gp - op[kk], 0.0)
                        res.append(jnp.maximum(acc[kk], th * gs))
                    return tuple(res)

                ninf = jnp.full((_L,), -jnp.inf, jnp.float32)
                acc = lax.fori_loop(0, n, nb, (ninf,) * nv)
                for kk in range(nv):
                    ctr = own[r, pl.ds((nv + kk) * _L, _L)]
                    outv[r, pl.ds(kk * _L, _L)] = jnp.maximum(ctr + acc[kk], 0.0)
                for kk in range(nv, wout // _L):
                    outv[r, pl.ds(kk * _L, _L)] = jnp.zeros((_L,), jnp.float32)
            pltpu.sync_copy(outv, out_hbm.at[pl.ds(base, c)])
            return 0

        lax.fori_loop(0, rows_per // c, chunk, 0)

    return k(tg, gidx, ownt)


def _sc_pool(table, gidx, *, n, nv, out_rows, c):
    w = nv * _L
    wt = table.shape[1]
    rows_per = out_rows // _NW

    @functools.partial(
        pl.kernel, mesh=_MESH,
        out_type=jax.ShapeDtypeStruct((out_rows, w), jnp.float32),
        scratch_types=[pltpu.VMEM((c * n,), jnp.int32),
                       pltpu.VMEM((c * n, wt), jnp.float32),
                       pltpu.VMEM((c, w), jnp.float32),
                       pltpu.SemaphoreType.DMA])
    def k(t_hbm, gidx_hbm, out_hbm, idxv, gath, outv, sem):
        base0 = _wid() * rows_per

        def chunk(ci, _):
            base = base0 + ci * c
            pltpu.sync_copy(gidx_hbm.at[pl.ds(base * n, c * n)], idxv)
            pltpu.async_copy(t_hbm.at[idxv], gath, sem).wait()
            for r in range(c):
                def nb(j, acc):
                    row = r * n + j
                    return tuple(
                        jnp.maximum(acc[kk], gath[row, pl.ds(kk * _L, _L)])
                        for kk in range(nv))

                ninf = jnp.full((_L,), -jnp.inf, jnp.float32)
                acc = lax.fori_loop(0, n, nb, (ninf,) * nv)
                for kk in range(nv):
                    outv[r, pl.ds(kk * _L, _L)] = acc[kk]
            pltpu.sync_copy(outv, out_hbm.at[pl.ds(base, c)])
            return 0

        lax.fori_loop(0, rows_per // c, chunk, 0)

    return k(table, gidx)


def kernel(vertices, w0, d0, w1, b1, disp1, w2, b2, disp2):
    b, v, _ = vertices.shape
    nn = NEIGHBOR_NUM

    idx21 = _knn_topk(vertices, nn + 1, 256)
    nidx = idx21[:, :, 1:]
    offs = (jnp.arange(b, dtype=jnp.int32) * v)[:, None, None]
    gidx1 = (nidx + offs).reshape(-1)

    vf = vertices.reshape(b * v, 3)
    tb, p64 = _dense1(vf, d0, disp1)
    fm0 = _sc_fm0(tb, gidx1, w0.reshape(-1))
    tg1, own1 = _dense2(p64, fm0, w1, b1.reshape(1, -1), 64)
    fm1 = _sc_edge(tg1, gidx1, own1, nv=4)

    p = v // 8
    sample_idx = jax.random.permutation(jax.random.key(101), v)[:p]
    gidx_e = (nidx[:, :, :8][:, sample_idx] + offs).reshape(-1)
    fm1p = _sc_pool(fm1, gidx_e, n=8, nv=4, out_rows=b * p, c=16)
    v1 = vertices[:, sample_idx]

    nidx2 = _knn_topk(v1, nn + 1, p)[:, :, 1:]
    offs2 = (jnp.arange(b, dtype=jnp.int32) * p)[:, None, None]
    gidx2 = (nidx2 + offs2).reshape(-1)
    v1f = v1.reshape(b * p, 3)

    def _p2_body(v_ref, d_ref, o_ref):
        o_ref[...] = _mm(v_ref[...], d_ref[...])

    p2t = pl.pallas_call(
        _p2_body, grid=(1,),
        in_specs=[pl.BlockSpec((b * p, 3), lambda i: (0, 0)),
                  pl.BlockSpec((3, 128), lambda i: (0, 0))],
        out_specs=pl.BlockSpec((b * p, 128), lambda i: (0, 0)),
        out_shape=jax.ShapeDtypeStruct((b * p, 128), jnp.float32),
    )(v1f, disp2)
    tg2, own2 = _dense2(p2t, fm1p, w2, b2.reshape(1, -1), 128)
    fm2 = _sc_edge(tg2, gidx2, own2, nv=8)

    p2 = p // 16
    sample2 = jax.random.permutation(jax.random.key(202), p)[:p2]
    gidx_g = (nidx2[:, :, :16][:, sample2] + offs2).reshape(-1)
    fm3 = _sc_pool(fm2, gidx_g, n=16, nv=8, out_rows=b * p2, c=8)
    v2 = v1[:, sample2]
    return (v2, fm3.reshape(b, p2, -1))


# speedup vs baseline: 20.4233x; 1.4900x over previous
import functools

import jax
import jax.numpy as jnp
from jax import lax
from jax.experimental import pallas as pl
from jax.experimental.pallas import tpu as pltpu
from jax.experimental.pallas import tpu_sc as plsc

NEIGHBOR_NUM = 20

_NC, _NS, _L = 2, 16, 16
_NW = _NC * _NS

_MESH = plsc.VectorSubcoreMesh(core_axis_name="c", subcore_axis_name="s")


def _wid():
    return lax.axis_index("s") * _NC + lax.axis_index("c")


def _topk_body(v_ref, vt_ref, out_ref, dist_ref, *, n, k):
    rows = v_ref.shape[1]
    vr = v_ref[0]
    vt = vt_ref[0]
    inner = lax.dot_general(
        vr, vt, (((1,), (0,)), ((), ())), preferred_element_type=jnp.float32)
    qr = jnp.sum(vr * vr, axis=1, keepdims=True)
    qc = jnp.sum(vt * vt, axis=0, keepdims=True)
    dist_ref[...] = inner * (-2.0) + qc + qr
    col = lax.broadcasted_iota(jnp.int32, (rows, n), 1)
    for t in range(k):
        dist = dist_ref[...]
        m = jnp.min(dist, axis=1, keepdims=True)
        idxt = jnp.min(jnp.where(dist == m, col, n), axis=1, keepdims=True)
        out_ref[0, :, t:t + 1] = idxt
        dist_ref[...] = jnp.where(col == idxt, jnp.inf, dist)


def _knn_topk(vertices, k, rows):
    b, n, _ = vertices.shape
    vt = jnp.transpose(vertices, (0, 2, 1))
    return pl.pallas_call(
        functools.partial(_topk_body, n=n, k=k),
        grid=(b, n // rows),
        in_specs=[
            pl.BlockSpec((1, rows, 3), lambda bi, ri: (bi, ri, 0)),
            pl.BlockSpec((1, 3, n), lambda bi, ri: (bi, 0, 0)),
        ],
        out_specs=pl.BlockSpec((1, rows, k), lambda bi, ri: (bi, ri, 0)),
        out_shape=jax.ShapeDtypeStruct((b, n, k), jnp.int32),
        scratch_shapes=[pltpu.VMEM((rows, n), jnp.float32)],
    )(vertices, vt)


def _mm(a, b):
    return lax.dot_general(a, b, (((1,), (0,)), ((), ())),
                           preferred_element_type=jnp.float32)


def _dense1(vf, d0, disp1):
    r = vf.shape[0]
    blk = 2048

    def body(v_ref, d0_ref, dp_ref, tb_ref, p64_ref):
        v = v_ref[...]
        p64 = _mm(v, dp_ref[...])
        tb_ref[:, :32] = _mm(v, d0_ref[...])
        tb_ref[:, 32:96] = p64
        tb_ref[:, 96:] = jnp.zeros((blk, 32), jnp.float32)
        p64_ref[...] = p64

    return pl.pallas_call(
        body, grid=(r // blk,),
        in_specs=[pl.BlockSpec((blk, 3), lambda i: (i, 0)),
                  pl.BlockSpec((3, 32), lambda i: (0, 0)),
                  pl.BlockSpec((3, 64), lambda i: (0, 0))],
        out_specs=[pl.BlockSpec((blk, 128), lambda i: (i, 0)),
                   pl.BlockSpec((blk, 64), lambda i: (i, 0))],
        out_shape=[jax.ShapeDtypeStruct((r, 128), jnp.float32),
                   jax.ShapeDtypeStruct((r, 64), jnp.float32)],
    )(vf, d0, disp1)


def _dense2(p, fm, w, bvec, oc):
    r = p.shape[0]
    blk = min(r, 2048)
    wname = 2 * oc

    def body(p_ref, f_ref, w_ref, b_ref, tg_ref, own_ref):
        fout = _mm(f_ref[...], w_ref[...]) + b_ref[...]
        pv = p_ref[...]
        tg_ref[:, :oc] = pv
        tg_ref[:, oc:] = fout[:, oc:]
        own_ref[:, :oc] = pv
        own_ref[:, oc:] = fout[:, :oc]

    return pl.pallas_call(
        body, grid=(r // blk,),
        in_specs=[pl.BlockSpec((blk, oc), lambda i: (i, 0)),
                  pl.BlockSpec((blk, fm.shape[1]), lambda i: (i, 0)),
                  pl.BlockSpec((fm.shape[1], wname), lambda i: (0, 0)),
                  pl.BlockSpec((1, wname), lambda i: (0, 0))],
        out_specs=[pl.BlockSpec((blk, wname), lambda i: (i, 0)),
                   pl.BlockSpec((blk, wname), lambda i: (i, 0))],
        out_shape=[jax.ShapeDtypeStruct((r, wname), jnp.float32),
                   jax.ShapeDtypeStruct((r, wname), jnp.float32)],
    )(p, fm, w, bvec)


def _sc_fm0(t0, gidx3, w0v):
    r_tot = t0.shape[0]
    n = NEIGHBOR_NUM
    nw, nchunks, nsub, _ = gidx3.shape
    c = nsub * 80 // n
    rows_per = r_tot // _NW

    @functools.partial(
        pl.kernel, mesh=_MESH,
        out_type=jax.ShapeDtypeStruct((r_tot, 32), jnp.float32),
        scratch_types=[pltpu.VMEM((nchunks, nsub, 80), jnp.int32),
                       pltpu.VMEM((c * n, 128), jnp.float32),
                       pltpu.VMEM((c * n, 128), jnp.float32),
                       pltpu.VMEM((c, 128), jnp.float32),
                       pltpu.VMEM((c, 128), jnp.float32),
                       pltpu.VMEM((c, 32), jnp.float32),
                       pltpu.VMEM((c, 32), jnp.float32),
                       pltpu.VMEM((32,), jnp.float32),
                       pltpu.SemaphoreType.DMA, pltpu.SemaphoreType.DMA,
                       pltpu.SemaphoreType.DMA, pltpu.SemaphoreType.DMA,
                       pltpu.SemaphoreType.DMA, pltpu.SemaphoreType.DMA])
    def k(t_hbm, gidx_hbm, w0_hbm, out_hbm, idxs, g0b, g1b, o0b, o1b,
          v0b, v1b, w0s, gs0, gs1, os0, os1, ss0, ss1):
        gath = (g0b, g1b)
        own = (o0b, o1b)
        outv = (v0b, v1b)
        gsem = (gs0, gs1)
        osem = (os0, os1)
        ssem = (ss0, ss1)
        wid = _wid()
        base0 = wid * rows_per
        pltpu.sync_copy(w0_hbm, w0s)
        pltpu.sync_copy(gidx_hbm.at[wid], idxs)
        w0a = w0s[pl.ds(0, _L)]
        w0b_ = w0s[pl.ds(_L, _L)]

        def start(ci, ph):
            for s in range(nsub):
                pltpu.make_async_copy(
                    t_hbm.at[idxs.at[ci, s]],
                    gath[ph].at[pl.ds(s * 80, 80)], gsem[ph]).start()
            pltpu.make_async_copy(
                t_hbm.at[pl.ds(base0 + ci * c, c)], own[ph], osem[ph]).start()

        start(0, 0)

        def chunk2(cj, _):
            for ph in range(2):
                ci = cj * 2 + ph

                @pl.when(ci + 1 < nchunks)
                def _():
                    start(ci + 1, 1 - ph)

                pltpu.make_async_copy(
                    t_hbm.at[pl.ds(0, c * n)], gath[ph], gsem[ph]).wait()
                pltpu.make_async_copy(
                    t_hbm.at[pl.ds(0, c)], own[ph], osem[ph]).wait()

                @pl.when(cj > 0)
                def _():
                    pltpu.make_async_copy(
                        outv[ph], out_hbm.at[pl.ds(base0, c)], ssem[ph]).wait()

                for r in range(c):
                    o0 = own[ph][r, pl.ds(0, _L)]
                    o1 = own[ph][r, pl.ds(_L, _L)]

                    def nb(j, acc):
                        row = r * n + j
                        ga = gath[ph][row, pl.ds(0, _L)]
                        gb = gath[ph][row, pl.ds(_L, _L)]
                        return (jnp.maximum(acc[0], jnp.maximum(ga - o0, 0.0)),
                                jnp.maximum(acc[1], jnp.maximum(gb - o1, 0.0)))

                    z = jnp.zeros((_L,), jnp.float32)
                    a0, a1 = lax.fori_loop(0, n, nb, (z, z))
                    outv[ph][r, pl.ds(0, _L)] = jnp.maximum(a0 * w0a, 0.0)
                    outv[ph][r, pl.ds(_L, _L)] = jnp.maximum(a1 * w0b_, 0.0)
                pltpu.make_async_copy(
                    outv[ph], out_hbm.at[pl.ds(base0 + ci * c, c)],
                    ssem[ph]).start()
            return 0

        lax.fori_loop(0, nchunks // 2, chunk2, 0)
        for ph in range(2):
            pltpu.make_async_copy(
                outv[ph], out_hbm.at[pl.ds(base0, c)], ssem[ph]).wait()

    return k(t0, gidx3, w0v)


def _sc_edge(tg, gidx3, ownt, *, nv):
    r_tot = tg.shape[0]
    w = 2 * nv * _L
    wout = max(nv * _L, 128)
    n = NEIGHBOR_NUM
    nw, nchunks, nsub, _ = gidx3.shape
    c = nsub * 80 // n
    rows_per = r_tot // _NW

    @functools.partial(
        pl.kernel, mesh=_MESH,
        out_type=jax.ShapeDtypeStruct((r_tot, wout), jnp.float32),
        scratch_types=[pltpu.VMEM((nchunks, nsub, 80), jnp.int32),
                       pltpu.VMEM((c * n, w), jnp.float32),
                       pltpu.VMEM((c * n, w), jnp.float32),
                       pltpu.VMEM((c, w), jnp.float32),
                       pltpu.VMEM((c, w), jnp.float32),
                       pltpu.VMEM((c, wout), jnp.float32),
                       pltpu.VMEM((c, wout), jnp.float32),
                       pltpu.SemaphoreType.DMA, pltpu.SemaphoreType.DMA,
                       pltpu.SemaphoreType.DMA, pltpu.SemaphoreType.DMA,
                       pltpu.SemaphoreType.DMA, pltpu.SemaphoreType.DMA])
    def k(tg_hbm, gidx_hbm, ownt_hbm, out_hbm, idxs, g0b, g1b, o0b, o1b,
          v0b, v1b, gs0, gs1, os0, os1, ss0, ss1):
        gath = (g0b, g1b)
        own = (o0b, o1b)
        outv = (v0b, v1b)
        gsem = (gs0, gs1)
        osem = (os0, os1)
        ssem = (ss0, ss1)
        wid = _wid()
        base0 = wid * rows_per
        pltpu.sync_copy(gidx_hbm.at[wid], idxs)

        def start(ci, ph):
            for s in range(nsub):
                pltpu.make_async_copy(
                    tg_hbm.at[idxs.at[ci, s]],
                    gath[ph].at[pl.ds(s * 80, 80)], gsem[ph]).start()
            pltpu.make_async_copy(
                ownt_hbm.at[pl.ds(base0 + ci * c, c)], own[ph],
                osem[ph]).start()

        start(0, 0)

        def chunk2(cj, _):
            for ph in range(2):
                ci = cj * 2 + ph

                @pl.when(ci + 1 < nchunks)
                def _():
                    start(ci + 1, 1 - ph)

                pltpu.make_async_copy(
                    tg_hbm.at[pl.ds(0, c * n)], gath[ph], gsem[ph]).wait()
                pltpu.make_async_copy(
                    ownt_hbm.at[pl.ds(0, c)], own[ph], osem[ph]).wait()

                @pl.when(cj > 0)
                def _():
                    pltpu.make_async_copy(
                        outv[ph], out_hbm.at[pl.ds(base0, c)], ssem[ph]).wait()

                for r in range(c):
                    op = [own[ph][r, pl.ds(kk * _L, _L)] for kk in range(nv)]

                    def nb(j, acc):
                        row = r * n + j
                        res = []
                        for kk in range(nv):
                            gp = gath[ph][row, pl.ds(kk * _L, _L)]
                            gs = gath[ph][row, pl.ds((nv + kk) * _L, _L)]
                            th = jnp.maximum(gp - op[kk], 0.0)
                            res.append(jnp.maximum(acc[kk], th * gs))
                        return tuple(res)

                    ninf = jnp.full((_L,), -jnp.inf, jnp.float32)
                    acc = lax.fori_loop(0, n, nb, (ninf,) * nv)
                    for kk in range(nv):
                        ctr = own[ph][r, pl.ds((nv + kk) * _L, _L)]
                        outv[ph][r, pl.ds(kk * _L, _L)] = jnp.maximum(
                            ctr + acc[kk], 0.0)
                    for kk in range(nv, wout // _L):
                        outv[ph][r, pl.ds(kk * _L, _L)] = jnp.zeros(
                            (_L,), jnp.float32)
                pltpu.make_async_copy(
                    outv[ph], out_hbm.at[pl.ds(base0 + ci * c, c)],
                    ssem[ph]).start()
            return 0

        lax.fori_loop(0, nchunks // 2, chunk2, 0)
        for ph in range(2):
            pltpu.make_async_copy(
                outv[ph], out_hbm.at[pl.ds(base0, c)], ssem[ph]).wait()

    return k(tg, gidx3, ownt)


def _sc_pool(table, gidx, *, n, nv, out_rows, c):
    w = nv * _L
    wt = table.shape[1]
    rows_per = out_rows // _NW

    @functools.partial(
        pl.kernel, mesh=_MESH,
        out_type=jax.ShapeDtypeStruct((out_rows, w), jnp.float32),
        scratch_types=[pltpu.VMEM((c * n,), jnp.int32),
                       pltpu.VMEM((c * n, wt), jnp.float32),
                       pltpu.VMEM((c, w), jnp.float32),
                       pltpu.SemaphoreType.DMA])
    def k(t_hbm, gidx_hbm, out_hbm, idxv, gath, outv, sem):
        base0 = _wid() * rows_per

        def chunk(ci, _):
            base = base0 + ci * c
            pltpu.sync_copy(gidx_hbm.at[pl.ds(base * n, c * n)], idxv)
            pltpu.async_copy(t_hbm.at[idxv], gath, sem).wait()
            for r in range(c):
                def nb(j, acc):
                    row = r * n + j
                    return tuple(
                        jnp.maximum(acc[kk], gath[row, pl.ds(kk * _L, _L)])
                        for kk in range(nv))

                ninf = jnp.full((_L,), -jnp.inf, jnp.float32)
                acc = lax.fori_loop(0, n, nb, (ninf,) * nv)
                for kk in range(nv):
                    outv[r, pl.ds(kk * _L, _L)] = acc[kk]
            pltpu.sync_copy(outv, out_hbm.at[pl.ds(base, c)])
            return 0

        lax.fori_loop(0, rows_per // c, chunk, 0)

    return k(table, gidx)


def kernel(vertices, w0, d0, w1, b1, disp1, w2, b2, disp2):
    b, v, _ = vertices.shape
    nn = NEIGHBOR_NUM

    idx21 = _knn_topk(vertices, nn + 1, 256)
    nidx = idx21[:, :, 1:]
    offs = (jnp.arange(b, dtype=jnp.int32) * v)[:, None, None]
    gidx1 = (nidx + offs).reshape(_NW, -1, 4, 80)

    vf = vertices.reshape(b * v, 3)
    tb, p64 = _dense1(vf, d0, disp1)
    fm0 = _sc_fm0(tb, gidx1, w0.reshape(-1))
    tg1, own1 = _dense2(p64, fm0, w1, b1.reshape(1, -1), 64)
    fm1 = _sc_edge(tg1, gidx1, own1, nv=4)

    p = v // 8
    sample_idx = jax.random.permutation(jax.random.key(101), v)[:p]
    gidx_e = (nidx[:, :, :8][:, sample_idx] + offs).reshape(-1)
    fm1p = _sc_pool(fm1, gidx_e, n=8, nv=4, out_rows=b * p, c=16)
    v1 = vertices[:, sample_idx]

    nidx2 = _knn_topk(v1, nn + 1, p)[:, :, 1:]
    offs2 = (jnp.arange(b, dtype=jnp.int32) * p)[:, None, None]
    gidx2 = (nidx2 + offs2).reshape(_NW, -1, 2, 80)
    v1f = v1.reshape(b * p, 3)

    def _p2_body(v_ref, d_ref, o_ref):
        o_ref[...] = _mm(v_ref[...], d_ref[...])

    p2t = pl.pallas_call(
        _p2_body, grid=(1,),
        in_specs=[pl.BlockSpec((b * p, 3), lambda i: (0, 0)),
                  pl.BlockSpec((3, 128), lambda i: (0, 0))],
        out_specs=pl.BlockSpec((b * p, 128), lambda i: (0, 0)),
        out_shape=jax.ShapeDtypeStruct((b * p, 128), jnp.float32),
    )(v1f, disp2)
    tg2, own2 = _dense2(p2t, fm1p, w2, b2.reshape(1, -1), 128)
    fm2 = _sc_edge(tg2, gidx2, own2, nv=8)

    p2 = p // 16
    sample2 = jax.random.permutation(jax.random.key(202), p)[:p2]
    gidx_g = (nidx2[:, :, :16][:, sample2] + offs2).reshape(-1)
    fm3 = _sc_pool(fm2, gidx_g, n=16, nv=8, out_rows=b * p2, c=8)
    v2 = v1[:, sample2]
    return (v2, fm3.reshape(b, p2, -1))
